# baseline (speedup 1.0000x reference)
import jax
import jax.numpy as jnp
from jax.experimental import pallas as pl
from jax.experimental.pallas import tpu as pltpu

_BS = 4096


def _fused_body(con_ref, c2_ref, c3_ref, c4_ref,
                e20_ref, e21_ref, e22_ref, e30_ref, e31_ref, e32_ref, e4_ref,
                W1_ref, b1_ref, W2_ref, b2_ref, W3_ref, b3_ref, out_ref):
    W1 = W1_ref[...]
    f32 = jnp.float32
    t20 = jnp.dot(e20_ref[...], W1[0:4], preferred_element_type=f32)
    t21 = jnp.dot(e21_ref[...], W1[4:8], preferred_element_type=f32)
    t22 = jnp.dot(e22_ref[...], W1[8:12], preferred_element_type=f32)
    t30 = jnp.dot(e30_ref[...], W1[12:18], preferred_element_type=f32)
    t31 = jnp.dot(e31_ref[...], W1[18:24], preferred_element_type=f32)
    t32 = jnp.dot(e32_ref[...], W1[24:30], preferred_element_type=f32)
    t4 = jnp.dot(e4_ref[...], W1[30:38], preferred_element_type=f32)

    wfold = jnp.concatenate([
        t20[1:2] - t20[0:1],
        t21[1:2] - t21[0:1],
        t22[1:2] - t22[0:1],
        t30[1:2] - t30[0:1],
        t31[1:2] - t31[0:1],
        t32[1:2] - t32[0:1],
        t4[1:2] - t4[0:1],
        t30[2:3] - t30[1:2],
        t31[2:3] - t31[1:2],
        t32[2:3] - t32[1:2],
        t4[2:3] - t4[1:2],
        t4[3:4] - t4[2:3],
        W1[38:44],
    ], axis=0)

    base = (b1_ref[...] + t20[0:1] + t21[0:1] + t22[0:1]
            + t30[0:1] + t31[0:1] + t32[0:1] + t4[0:1])

    call = jnp.concatenate(
        [c2_ref[...], c3_ref[...], c4_ref[...]], axis=1).astype(f32)
    s = jnp.concatenate([
        (call >= 1.0).astype(f32),
        (call[:, 3:7] >= 2.0).astype(f32),
        (call[:, 6:7] >= 3.0).astype(f32),
        con_ref[...],
    ], axis=1)

    h = jnp.tanh(jnp.dot(s, wfold, preferred_element_type=f32) + base)
    h = jnp.tanh(jnp.dot(h, W2_ref[...], preferred_element_type=f32)
                 + b2_ref[...])
    y = jnp.dot(h, W3_ref[...], preferred_element_type=f32) + b3_ref[...]
    out_ref[...] = jax.nn.sigmoid(y)


def kernel(con_x, cat_2, cat_3, cat_4,
           emb2_0, emb2_1, emb2_2, emb3_0, emb3_1, emb3_2, emb4,
           W1, b1, W2, b2, W3, b3):
    b1r = b1.reshape(1, -1)
    b2r = b2.reshape(1, -1)
    b3r = b3.reshape(1, -1)
    c2 = cat_2.astype(jnp.int32)
    c3 = cat_3.astype(jnp.int32)
    c4 = cat_4.astype(jnp.int32)

    B = con_x.shape[0]
    grid = (B // _BS,)

    def full(shape):
        nd = len(shape)
        return pl.BlockSpec(shape, lambda i: (0,) * nd)

    def row(width):
        return pl.BlockSpec((_BS, width), lambda i: (i, 0))

    out = pl.pallas_call(
        _fused_body,
        grid=grid,
        in_specs=[
            row(6), row(3), row(3), row(1),
            full(emb2_0.shape), full(emb2_1.shape), full(emb2_2.shape),
            full(emb3_0.shape), full(emb3_1.shape), full(emb3_2.shape),
            full(emb4.shape),
            full(W1.shape), full(b1r.shape),
            full(W2.shape), full(b2r.shape),
            full(W3.shape), full(b3r.shape),
        ],
        out_specs=pl.BlockSpec((_BS, 2), lambda i: (i, 0)),
        out_shape=jax.ShapeDtypeStruct((B, 2), jnp.float32),
        compiler_params=pltpu.CompilerParams(
            dimension_semantics=("arbitrary",),
        ),
    )(con_x, c2, c3, c4,
      emb2_0, emb2_1, emb2_2, emb3_0, emb3_1, emb3_2, emb4,
      W1, b1r, W2, b2r, W3, b3r)
    return out


# speedup vs baseline: 1.1486x; 1.1486x over previous
import jax
import jax.numpy as jnp
import numpy as np
from jax.experimental import pallas as pl
from jax.experimental.pallas import tpu as pltpu

_BS = 4096

_PATTERN = np.array([0, 1, 0, 1, 0, 1,
                     0, 1, 2, 0, 1, 2, 0, 1, 2,
                     0, 1, 2, 3] + [-1] * 6, dtype=np.float32)[None, :]

_WROWS = 451


def _fused_body(x_ref, w_ref, out_ref):
    f32 = jnp.float32
    W1 = w_ref[0:44, :]
    wfold = jnp.concatenate([
        jnp.dot(w_ref[432:434, 0:4], W1[0:4], preferred_element_type=f32),
        jnp.dot(w_ref[434:436, 0:4], W1[4:8], preferred_element_type=f32),
        jnp.dot(w_ref[436:438, 0:4], W1[8:12], preferred_element_type=f32),
        jnp.dot(w_ref[438:441, 0:6], W1[12:18], preferred_element_type=f32),
        jnp.dot(w_ref[441:444, 0:6], W1[18:24], preferred_element_type=f32),
        jnp.dot(w_ref[444:447, 0:6], W1[24:30], preferred_element_type=f32),
        jnp.dot(w_ref[447:451, 0:8], W1[30:38], preferred_element_type=f32),
        W1[38:44],
    ], axis=0)

    x = x_ref[...]
    lane = jax.lax.broadcasted_iota(jnp.int32, x.shape, 1)
    x = jnp.where(lane < 19, (x == w_ref[431:432, 0:25]).astype(f32), x)

    h = jnp.tanh(jnp.dot(x, wfold, preferred_element_type=f32)
                 + w_ref[428:429, :])
    h = jnp.tanh(jnp.dot(h, w_ref[44:300, 0:128], preferred_element_type=f32)
                 + w_ref[429:430, 0:128])
    y = (jnp.dot(h, w_ref[300:428, 0:2], preferred_element_type=f32)
         + w_ref[430:431, 0:2])
    out_ref[...] = jax.nn.sigmoid(y)


def _pad256(a):
    return jnp.pad(a, ((0, 0), (0, 256 - a.shape[1])))


def kernel(con_x, cat_2, cat_3, cat_4,
           emb2_0, emb2_1, emb2_2, emb3_0, emb3_1, emb3_2, emb4,
           W1, b1, W2, b2, W3, b3):
    x_packed = jnp.concatenate([
        jnp.repeat(cat_2.astype(jnp.float32), 2, axis=1),
        jnp.repeat(cat_3.astype(jnp.float32), 3, axis=1),
        jnp.repeat(cat_4.astype(jnp.float32), 4, axis=1),
        con_x,
    ], axis=1)

    wpack = jnp.concatenate([
        W1,
        _pad256(W2),
        _pad256(W3),
        b1.reshape(1, -1),
        _pad256(b2.reshape(1, -1)),
        _pad256(b3.reshape(1, -1)),
        _pad256(jnp.asarray(_PATTERN)),
        _pad256(emb2_0), _pad256(emb2_1), _pad256(emb2_2),
        _pad256(emb3_0), _pad256(emb3_1), _pad256(emb3_2),
        _pad256(emb4),
    ], axis=0)

    B = con_x.shape[0]
    grid = (B // _BS,)

    out = pl.pallas_call(
        _fused_body,
        grid=grid,
        in_specs=[
            pl.BlockSpec((_BS, 25), lambda i: (i, 0)),
            pl.BlockSpec((_WROWS, 256), lambda i: (0, 0)),
        ],
        out_specs=pl.BlockSpec((_BS, 2), lambda i: (i, 0)),
        out_shape=jax.ShapeDtypeStruct((B, 2), jnp.float32),
        compiler_params=pltpu.CompilerParams(
            dimension_semantics=("arbitrary",),
        ),
    )(x_packed, wpack)
    return out


# speedup vs baseline: 1.5089x; 1.3137x over previous
import jax
import jax.numpy as jnp
import numpy as np
from jax.experimental import pallas as pl
from jax.experimental.pallas import tpu as pltpu

_BS = 4096

_THRESH = np.array([1, 1, 1, 1, 1, 1, 1, 2, 2, 2, 2, 3] + [99] * 6,
                   dtype=np.float32)[None, :]


def _fused_body(x_ref, thr_ref,
                e20_ref, e21_ref, e22_ref, e30_ref, e31_ref, e32_ref, e4_ref,
                W1_ref, b1_ref, W2_ref, b2_ref, W3_ref, b3_ref, out_ref):
    W1 = W1_ref[...]
    f32 = jnp.float32
    t20 = jnp.dot(e20_ref[...], W1[0:4], preferred_element_type=f32)
    t21 = jnp.dot(e21_ref[...], W1[4:8], preferred_element_type=f32)
    t22 = jnp.dot(e22_ref[...], W1[8:12], preferred_element_type=f32)
    t30 = jnp.dot(e30_ref[...], W1[12:18], preferred_element_type=f32)
    t31 = jnp.dot(e31_ref[...], W1[18:24], preferred_element_type=f32)
    t32 = jnp.dot(e32_ref[...], W1[24:30], preferred_element_type=f32)
    t4 = jnp.dot(e4_ref[...], W1[30:38], preferred_element_type=f32)

    wfold = jnp.concatenate([
        t20[1:2] - t20[0:1],
        t21[1:2] - t21[0:1],
        t22[1:2] - t22[0:1],
        t30[1:2] - t30[0:1],
        t31[1:2] - t31[0:1],
        t32[1:2] - t32[0:1],
        t4[1:2] - t4[0:1],
        t30[2:3] - t30[1:2],
        t31[2:3] - t31[1:2],
        t32[2:3] - t32[1:2],
        t4[2:3] - t4[1:2],
        t4[3:4] - t4[2:3],
        W1[38:44],
    ], axis=0)

    base = (b1_ref[...] + t20[0:1] + t21[0:1] + t22[0:1]
            + t30[0:1] + t31[0:1] + t32[0:1] + t4[0:1])

    x = x_ref[...]
    lane = jax.lax.broadcasted_iota(jnp.int32, x.shape, 1)
    s = jnp.where(lane < 12, (x >= thr_ref[...]).astype(f32), x)

    h = jnp.tanh(jnp.dot(s, wfold, preferred_element_type=f32) + base)
    h = jnp.tanh(jnp.dot(h, W2_ref[...], preferred_element_type=f32)
                 + b2_ref[...])
    y = jnp.dot(h, W3_ref[...], preferred_element_type=f32) + b3_ref[...]
    out_ref[...] = jax.nn.sigmoid(y)


def kernel(con_x, cat_2, cat_3, cat_4,
           emb2_0, emb2_1, emb2_2, emb3_0, emb3_1, emb3_2, emb4,
           W1, b1, W2, b2, W3, b3):
    c2 = cat_2.astype(jnp.float32)
    c3 = cat_3.astype(jnp.float32)
    c4 = cat_4.astype(jnp.float32)
    x_packed = jnp.concatenate([c2, c3, c4, c3, c4, c4, con_x], axis=1)

    b1r = b1.reshape(1, -1)
    b2r = b2.reshape(1, -1)
    b3r = b3.reshape(1, -1)

    B = con_x.shape[0]
    grid = (B // _BS,)

    def full(shape):
        nd = len(shape)
        return pl.BlockSpec(shape, lambda i: (0,) * nd)

    out = pl.pallas_call(
        _fused_body,
        grid=grid,
        in_specs=[
            pl.BlockSpec((_BS, 18), lambda i: (i, 0)),
            pl.BlockSpec((1, 18), lambda i: (0, 0)),
            full(emb2_0.shape), full(emb2_1.shape), full(emb2_2.shape),
            full(emb3_0.shape), full(emb3_1.shape), full(emb3_2.shape),
            full(emb4.shape),
            full(W1.shape), full(b1r.shape),
            full(W2.shape), full(b2r.shape),
            full(W3.shape), full(b3r.shape),
        ],
        out_specs=pl.BlockSpec((_BS, 2), lambda i: (i, 0)),
        out_shape=jax.ShapeDtypeStruct((B, 2), jnp.float32),
        compiler_params=pltpu.CompilerParams(
            dimension_semantics=("arbitrary",),
        ),
    )(x_packed, jnp.asarray(_THRESH),
      emb2_0, emb2_1, emb2_2, emb3_0, emb3_1, emb3_2, emb4,
      W1, b1r, W2, b2r, W3, b3r)
    return out
